# Initial kernel scaffold; baseline (speedup 1.0000x reference)
import jax
import jax.numpy as jnp
from jax.experimental import pallas as pl


def kernel(x, edge_index, W1, b1, W2, b2, W3, b3):
    raise NotImplementedError("write your pallas kernel here")


# speedup vs baseline: 5.3032x; 5.3032x over previous
import functools

import jax
import jax.numpy as jnp
from jax import lax
from jax.experimental import pallas as pl
from jax.experimental.pallas import tpu as pltpu
from jax.experimental.pallas import tpu_sc as plsc

_NPAD = 10240
_ECHUNK = 128
_NB = 1024


def _sc_degree(dst2d, npad):
    nch = dst2d.shape[0]
    ch_per_tile = nch // 32
    rpt = npad // 16
    w = 16
    zrows = 64

    mesh = plsc.VectorSubcoreMesh(core_axis_name="c", subcore_axis_name="s")

    @functools.partial(
        pl.kernel,
        out_type=jax.ShapeDtypeStruct((2 * npad, w), jnp.float32),
        mesh=mesh,
        compiler_params=pltpu.CompilerParams(use_tc_tiling_on_sc=False),
        scratch_types=[
            pltpu.VMEM((128,), jnp.int32),
            pltpu.VMEM((_ECHUNK, w), jnp.float32),
            pltpu.VMEM((zrows, w), jnp.float32),
            pltpu.VMEM_SHARED((npad, w), jnp.float32),
        ],
    )
    def k(dst_hbm, ones_hbm, zeros_hbm, deg_hbm, dstage, ones, zbuf, acc):
        c = lax.axis_index("c")
        s = lax.axis_index("s")
        wid = s * 2 + c

        pltpu.sync_copy(ones_hbm, ones)
        pltpu.sync_copy(zeros_hbm, zbuf)

        def zacc(r, _):
            pltpu.sync_copy(zbuf, acc.at[pl.ds(s * rpt + r * zrows, zrows)])
            return 0
        lax.fori_loop(0, rpt // zrows, zacc, 0)

        plsc.subcore_barrier()

        def body(j, _):
            pltpu.sync_copy(dst_hbm.at[wid * ch_per_tile + j], dstage)
            pltpu.sync_copy(ones, acc.at[dstage], add=True)
            return 0
        lax.fori_loop(0, ch_per_tile, body, 0)

        plsc.subcore_barrier()
        pltpu.sync_copy(acc.at[pl.ds(s * rpt, rpt)],
                        deg_hbm.at[pl.ds(c * npad + s * rpt, rpt)])

    return k(dst2d, jnp.ones((_ECHUNK, w), jnp.float32),
             jnp.zeros((zrows, w), jnp.float32))


def _sc_scatter(gflat, src2d, dst2d, npad, w):
    nch = src2d.shape[0]
    ch_per_tile = nch // 16
    rpt = npad // 16
    zrows = 64

    src_off = jnp.concatenate([src2d, src2d + npad], axis=0)

    mesh = plsc.VectorSubcoreMesh(core_axis_name="c", subcore_axis_name="s")

    @functools.partial(
        pl.kernel,
        out_type=jax.ShapeDtypeStruct((2 * npad, w), jnp.float32),
        mesh=mesh,
        compiler_params=pltpu.CompilerParams(use_tc_tiling_on_sc=False),
        scratch_types=[
            pltpu.VMEM((128,), jnp.int32),
            pltpu.VMEM((128,), jnp.int32),
            pltpu.VMEM((_ECHUNK, w), jnp.float32),
            pltpu.VMEM((zrows, w), jnp.float32),
            pltpu.VMEM_SHARED((npad, w), jnp.float32),
            pltpu.SemaphoreType.DMA,
        ],
    )
    def k(g_hbm, src_hbm, dst_hbm, zeros_hbm, out_hbm, sstage,
          dstage, buf, zbuf, acc, sem):
        c = lax.axis_index("c")
        s = lax.axis_index("s")

        pltpu.sync_copy(zeros_hbm, zbuf)

        def zacc(r, _):
            pltpu.sync_copy(zbuf, acc.at[pl.ds(s * rpt + r * zrows, zrows)])
            return 0
        lax.fori_loop(0, rpt // zrows, zacc, 0)

        plsc.subcore_barrier()

        sbase = c * nch + s * ch_per_tile
        dbase = s * ch_per_tile

        def body(j, _):
            pltpu.sync_copy(src_hbm.at[sbase + j], sstage)
            pltpu.sync_copy(dst_hbm.at[dbase + j], dstage)
            pltpu.async_copy(g_hbm.at[sstage], buf, sem).wait()
            pltpu.sync_copy(buf, acc.at[dstage], add=True)
            return 0
        lax.fori_loop(0, ch_per_tile, body, 0)

        plsc.subcore_barrier()

        pltpu.sync_copy(acc.at[pl.ds(s * rpt, rpt)],
                        out_hbm.at[pl.ds(c * npad + s * rpt, rpt)])

    return k(gflat, src_off, dst2d, jnp.zeros((zrows, w), jnp.float32))


def _sc_scatter_esplit(g, src2d, dst2d, npad):
    nch = src2d.shape[0]
    ch_per_tile = nch // 32
    rpt = npad // 16
    zrows = 64
    w = 128

    mesh = plsc.VectorSubcoreMesh(core_axis_name="c", subcore_axis_name="s")

    @functools.partial(
        pl.kernel,
        out_type=jax.ShapeDtypeStruct((2 * npad, w), jnp.float32),
        mesh=mesh,
        compiler_params=pltpu.CompilerParams(use_tc_tiling_on_sc=False),
        scratch_types=[
            pltpu.VMEM((128,), jnp.int32),
            pltpu.VMEM((128,), jnp.int32),
            pltpu.VMEM((_ECHUNK, w), jnp.float32),
            pltpu.VMEM((zrows, w), jnp.float32),
            pltpu.VMEM_SHARED((npad, w), jnp.float32),
            pltpu.SemaphoreType.DMA,
        ],
    )
    def k(g_hbm, src_hbm, dst_hbm, zeros_hbm, out_hbm, sstage,
          dstage, buf, zbuf, acc, sem):
        c = lax.axis_index("c")
        s = lax.axis_index("s")

        pltpu.sync_copy(zeros_hbm, zbuf)

        def zacc(r, _):
            pltpu.sync_copy(zbuf, acc.at[pl.ds(s * rpt + r * zrows, zrows)])
            return 0
        lax.fori_loop(0, rpt // zrows, zacc, 0)

        plsc.subcore_barrier()

        base = c * (nch // 2) + s * ch_per_tile

        def body(j, _):
            pltpu.sync_copy(src_hbm.at[base + j], sstage)
            pltpu.sync_copy(dst_hbm.at[base + j], dstage)
            pltpu.async_copy(g_hbm.at[sstage], buf, sem).wait()
            pltpu.sync_copy(buf, acc.at[dstage], add=True)
            return 0
        lax.fori_loop(0, ch_per_tile, body, 0)

        plsc.subcore_barrier()

        pltpu.sync_copy(acc.at[pl.ds(s * rpt, rpt)],
                        out_hbm.at[pl.ds(c * npad + s * rpt, rpt)])

    return k(g, src2d, dst2d, jnp.zeros((zrows, w), jnp.float32))


def _tc_layer1(xp, w1p, deg2, npad):
    h = w1p.shape[1]
    grid = (npad // _NB,)

    def body(x_ref, w_ref, deg_ref, g_ref, dinv_ref):
        d = deg_ref[0, :, 0:1] + deg_ref[1, :, 0:1] + 1.0
        dinv = lax.rsqrt(d)
        hh = jnp.dot(x_ref[...], w_ref[...], preferred_element_type=jnp.float32)
        g = hh * dinv
        g_ref[0] = g[:, : h // 2]
        g_ref[1] = g[:, h // 2:]
        dinv_ref[...] = dinv

    return pl.pallas_call(
        body,
        grid=grid,
        in_specs=[
            pl.BlockSpec((_NB, 128), lambda i: (i, 0)),
            pl.BlockSpec((128, h), lambda i: (0, 0)),
            pl.BlockSpec((2, _NB, 16), lambda i: (0, i, 0)),
        ],
        out_specs=[
            pl.BlockSpec((2, _NB, h // 2), lambda i: (0, i, 0)),
            pl.BlockSpec((_NB, 1), lambda i: (i, 0)),
        ],
        out_shape=[
            jax.ShapeDtypeStruct((2, npad, h // 2), jnp.float32),
            jax.ShapeDtypeStruct((npad, 1), jnp.float32),
        ],
    )(xp, w1p, deg2)


def _tc_mid(agg, g, dinv, b2d, wmat, npad, split_out):
    win = agg.shape[2]
    hout = wmat.shape[1]
    grid = (npad // _NB,)

    def body(agg_ref, g_ref, dinv_ref, b_ref, w_ref, out_ref):
        dinv = dinv_ref[...]
        b = b_ref[...]
        a0 = (agg_ref[0] + g_ref[0]) * dinv + b[:, :win]
        a1 = (agg_ref[1] + g_ref[1]) * dinv + b[:, win:]
        act = jnp.maximum(jnp.concatenate([a0, a1], axis=1), 0.0)
        hh = jnp.dot(act, w_ref[...], preferred_element_type=jnp.float32)
        gg = hh * dinv
        if split_out:
            out_ref[0] = gg[:, : hout // 2]
            out_ref[1] = gg[:, hout // 2:]
        else:
            out_ref[...] = gg

    if split_out:
        out_spec = pl.BlockSpec((2, _NB, hout // 2), lambda i: (0, i, 0))
        out_shape = jax.ShapeDtypeStruct((2, npad, hout // 2), jnp.float32)
    else:
        out_spec = pl.BlockSpec((_NB, hout), lambda i: (i, 0))
        out_shape = jax.ShapeDtypeStruct((npad, hout), jnp.float32)

    return pl.pallas_call(
        body,
        grid=grid,
        in_specs=[
            pl.BlockSpec((2, _NB, win), lambda i: (0, i, 0)),
            pl.BlockSpec((2, _NB, win), lambda i: (0, i, 0)),
            pl.BlockSpec((_NB, 1), lambda i: (i, 0)),
            pl.BlockSpec((1, 2 * win), lambda i: (0, 0)),
            pl.BlockSpec((2 * win, hout), lambda i: (0, 0)),
        ],
        out_specs=out_spec,
        out_shape=out_shape,
    )(agg, g, dinv, b2d, wmat)


def _tc_final(agg2, g, dinv, b2d, npad):
    w = g.shape[1]
    grid = (npad // _NB,)

    def body(agg_ref, g_ref, dinv_ref, b_ref, out_ref):
        dinv = dinv_ref[...]
        out_ref[...] = (agg_ref[0] + agg_ref[1] + g_ref[...]) * dinv + b_ref[...]

    return pl.pallas_call(
        body,
        grid=grid,
        in_specs=[
            pl.BlockSpec((2, _NB, w), lambda i: (0, i, 0)),
            pl.BlockSpec((_NB, w), lambda i: (i, 0)),
            pl.BlockSpec((_NB, 1), lambda i: (i, 0)),
            pl.BlockSpec((1, w), lambda i: (0, 0)),
        ],
        out_specs=pl.BlockSpec((_NB, w), lambda i: (i, 0)),
        out_shape=jax.ShapeDtypeStruct((npad, w), jnp.float32),
    )(agg2, g, dinv, b2d)


def kernel(x, edge_index, W1, b1, W2, b2, W3, b3):
    n, f = x.shape
    e = edge_index.shape[1]
    h = W1.shape[1]
    c_out = W3.shape[1]
    npad = _NPAD
    nch = -(-e // _ECHUNK)
    nch = -(-nch // 32) * 32
    epad = nch * _ECHUNK

    xp = jnp.zeros((npad, 128), x.dtype).at[:n, :f].set(x)
    w1p = jnp.zeros((128, h), W1.dtype).at[:f].set(W1)
    w3p = jnp.zeros((h, 128), W3.dtype).at[:, :c_out].set(W3)
    b3p = jnp.zeros((1, 128), b3.dtype).at[0, :c_out].set(b3)
    b1r = b1.reshape(1, h)
    b2r = b2.reshape(1, h)

    fill = jnp.full((epad - e,), npad - 1, jnp.int32)
    src2d = jnp.concatenate([edge_index[0], fill]).reshape(nch, 128)
    dst2d = jnp.concatenate([edge_index[1], fill]).reshape(nch, 128)

    deg2 = _sc_degree(dst2d, npad).reshape(2, npad, 16)
    g1, dinv = _tc_layer1(xp, w1p, deg2, npad)

    agg1 = _sc_scatter(g1.reshape(2 * npad, h // 2), src2d, dst2d, npad, h // 2)
    g2 = _tc_mid(agg1.reshape(2, npad, h // 2), g1, dinv, b1r, W2, npad,
                 split_out=True)

    agg2 = _sc_scatter(g2.reshape(2 * npad, h // 2), src2d, dst2d, npad, h // 2)
    g3 = _tc_mid(agg2.reshape(2, npad, h // 2), g2, dinv, b2r, w3p, npad,
                 split_out=False)

    agg3 = _sc_scatter_esplit(g3, src2d, dst2d, npad)
    out = _tc_final(agg3.reshape(2, npad, 128), g3, dinv, b3p, npad)

    return out[:n, :c_out]


# speedup vs baseline: 6.9418x; 1.3090x over previous
import functools

import jax
import jax.numpy as jnp
from jax import lax
from jax.experimental import pallas as pl
from jax.experimental.pallas import tpu as pltpu
from jax.experimental.pallas import tpu_sc as plsc

_NPAD = 10240
_ECHUNK = 128
_NB = 1024


def _sc_degree(dst2d, npad):
    nch = dst2d.shape[0]
    ch_per_tile = nch // 32
    rpt = npad // 16
    w = 16
    zrows = 64

    mesh = plsc.VectorSubcoreMesh(core_axis_name="c", subcore_axis_name="s")

    @functools.partial(
        pl.kernel,
        out_type=jax.ShapeDtypeStruct((2 * npad, w), jnp.float32),
        mesh=mesh,
        compiler_params=pltpu.CompilerParams(use_tc_tiling_on_sc=False),
        scratch_types=[
            pltpu.VMEM((ch_per_tile, 128), jnp.int32),
            pltpu.VMEM((_ECHUNK, w), jnp.float32),
            pltpu.VMEM((zrows, w), jnp.float32),
            pltpu.VMEM_SHARED((npad, w), jnp.float32),
        ],
    )
    def k(dst_hbm, ones_hbm, zeros_hbm, deg_hbm, didx, ones, zbuf, acc):
        c = lax.axis_index("c")
        s = lax.axis_index("s")
        wid = s * 2 + c

        pltpu.sync_copy(ones_hbm, ones)
        pltpu.sync_copy(zeros_hbm, zbuf)

        def zacc(r, _):
            pltpu.sync_copy(zbuf, acc.at[pl.ds(s * rpt + r * zrows, zrows)])
            return 0
        lax.fori_loop(0, rpt // zrows, zacc, 0)

        pltpu.sync_copy(dst_hbm.at[pl.ds(wid * ch_per_tile, ch_per_tile)],
                        didx)
        plsc.subcore_barrier()

        def body(j, _):
            pltpu.sync_copy(ones, acc.at[didx.at[j]], add=True)
            return 0
        lax.fori_loop(0, ch_per_tile, body, 0)

        plsc.subcore_barrier()
        pltpu.sync_copy(acc.at[pl.ds(s * rpt, rpt)],
                        deg_hbm.at[pl.ds(c * npad + s * rpt, rpt)])

    return k(dst2d, jnp.ones((_ECHUNK, w), jnp.float32),
             jnp.zeros((zrows, w), jnp.float32))


def _sc_scatter(gflat, src2d, dst2d, npad, w):
    nch = src2d.shape[0]
    cw = 64
    nch2 = nch * 2
    ch_per_tile = nch2 // 16
    rpt = npad // 16

    src_off = jnp.concatenate([src2d, src2d + npad], axis=0).reshape(
        2 * nch2, cw)
    dst2 = dst2d.reshape(nch2, cw)

    mesh = plsc.VectorSubcoreMesh(core_axis_name="c", subcore_axis_name="s")

    @functools.partial(
        pl.kernel,
        out_type=jax.ShapeDtypeStruct((2 * npad, w), jnp.float32),
        mesh=mesh,
        compiler_params=pltpu.CompilerParams(use_tc_tiling_on_sc=False),
        scratch_types=[
            pltpu.VMEM((ch_per_tile, cw), jnp.int32),
            pltpu.VMEM((ch_per_tile, cw), jnp.int32),
            pltpu.VMEM((cw, w), jnp.float32),
            pltpu.VMEM((cw, w), jnp.float32),
            pltpu.VMEM_SHARED((npad, w), jnp.float32),
            pltpu.SemaphoreType.DMA,
            pltpu.SemaphoreType.DMA,
        ],
    )
    def k(g_hbm, src_hbm, dst_hbm, zeros_hbm, out_hbm, sidx, didx,
          bufa, bufb, acc, sema, semb):
        c = lax.axis_index("c")
        s = lax.axis_index("s")

        pltpu.sync_copy(zeros_hbm, bufa)

        def zacc(r, _):
            pltpu.sync_copy(bufa, acc.at[pl.ds(s * rpt + r * cw, cw)])
            return 0
        lax.fori_loop(0, rpt // cw, zacc, 0)

        pltpu.sync_copy(src_hbm.at[pl.ds(c * nch2 + s * ch_per_tile,
                                         ch_per_tile)], sidx)
        pltpu.sync_copy(dst_hbm.at[pl.ds(s * ch_per_tile, ch_per_tile)], didx)

        plsc.subcore_barrier()

        pltpu.async_copy(g_hbm.at[sidx.at[0]], bufa, sema)

        def body(i, _):
            a = 2 * i
            b = a + 1
            nxt = jnp.minimum(a + 2, ch_per_tile - 1)
            pltpu.async_copy(g_hbm.at[sidx.at[b]], bufb, semb)
            pltpu.make_async_copy(g_hbm.at[sidx.at[a]], bufa, sema).wait()
            pltpu.sync_copy(bufa, acc.at[didx.at[a]], add=True)
            pltpu.async_copy(g_hbm.at[sidx.at[nxt]], bufa, sema)
            pltpu.make_async_copy(g_hbm.at[sidx.at[b]], bufb, semb).wait()
            pltpu.sync_copy(bufb, acc.at[didx.at[b]], add=True)
            return 0
        lax.fori_loop(0, ch_per_tile // 2, body, 0)
        pltpu.make_async_copy(g_hbm.at[sidx.at[ch_per_tile - 1]], bufa,
                              sema).wait()

        plsc.subcore_barrier()

        pltpu.sync_copy(acc.at[pl.ds(s * rpt, rpt)],
                        out_hbm.at[pl.ds(c * npad + s * rpt, rpt)])

    return k(gflat, src_off, dst2, jnp.zeros((cw, w), jnp.float32))


def _sc_scatter_esplit(g, src2d, dst2d, npad):
    nch = src2d.shape[0]
    cw = 64
    nch2 = nch * 2
    ch_per_tile = nch2 // 32
    rpt = npad // 16
    w = 128

    src2 = src2d.reshape(nch2, cw)
    dst2 = dst2d.reshape(nch2, cw)

    mesh = plsc.VectorSubcoreMesh(core_axis_name="c", subcore_axis_name="s")

    @functools.partial(
        pl.kernel,
        out_type=jax.ShapeDtypeStruct((2 * npad, w), jnp.float32),
        mesh=mesh,
        compiler_params=pltpu.CompilerParams(use_tc_tiling_on_sc=False),
        scratch_types=[
            pltpu.VMEM((ch_per_tile, cw), jnp.int32),
            pltpu.VMEM((ch_per_tile, cw), jnp.int32),
            pltpu.VMEM((cw, w), jnp.float32),
            pltpu.VMEM((cw, w), jnp.float32),
            pltpu.VMEM_SHARED((npad, w), jnp.float32),
            pltpu.SemaphoreType.DMA,
            pltpu.SemaphoreType.DMA,
        ],
    )
    def k(g_hbm, src_hbm, dst_hbm, zeros_hbm, out_hbm, sidx, didx,
          bufa, bufb, acc, sema, semb):
        c = lax.axis_index("c")
        s = lax.axis_index("s")

        pltpu.sync_copy(zeros_hbm, bufa)

        def zacc(r, _):
            pltpu.sync_copy(bufa, acc.at[pl.ds(s * rpt + r * cw, cw)])
            return 0
        lax.fori_loop(0, rpt // cw, zacc, 0)

        base = c * (nch2 // 2) + s * ch_per_tile
        pltpu.sync_copy(src_hbm.at[pl.ds(base, ch_per_tile)], sidx)
        pltpu.sync_copy(dst_hbm.at[pl.ds(base, ch_per_tile)], didx)

        plsc.subcore_barrier()

        pltpu.async_copy(g_hbm.at[sidx.at[0]], bufa, sema)

        def body(i, _):
            a = 2 * i
            b = a + 1
            nxt = jnp.minimum(a + 2, ch_per_tile - 1)
            pltpu.async_copy(g_hbm.at[sidx.at[b]], bufb, semb)
            pltpu.make_async_copy(g_hbm.at[sidx.at[a]], bufa, sema).wait()
            pltpu.sync_copy(bufa, acc.at[didx.at[a]], add=True)
            pltpu.async_copy(g_hbm.at[sidx.at[nxt]], bufa, sema)
            pltpu.make_async_copy(g_hbm.at[sidx.at[b]], bufb, semb).wait()
            pltpu.sync_copy(bufb, acc.at[didx.at[b]], add=True)
            return 0
        lax.fori_loop(0, ch_per_tile // 2, body, 0)
        pltpu.make_async_copy(g_hbm.at[sidx.at[ch_per_tile - 1]], bufa,
                              sema).wait()

        plsc.subcore_barrier()

        pltpu.sync_copy(acc.at[pl.ds(s * rpt, rpt)],
                        out_hbm.at[pl.ds(c * npad + s * rpt, rpt)])

    return k(g, src2, dst2, jnp.zeros((cw, w), jnp.float32))


def _tc_layer1(xp, w1p, deg2, npad):
    h = w1p.shape[1]
    grid = (npad // _NB,)

    def body(x_ref, w_ref, deg_ref, g_ref, dinv_ref):
        d = deg_ref[0, :, 0:1] + deg_ref[1, :, 0:1] + 1.0
        dinv = lax.rsqrt(d)
        hh = jnp.dot(x_ref[...], w_ref[...], preferred_element_type=jnp.float32)
        g = hh * dinv
        g_ref[0] = g[:, : h // 2]
        g_ref[1] = g[:, h // 2:]
        dinv_ref[...] = dinv

    return pl.pallas_call(
        body,
        grid=grid,
        in_specs=[
            pl.BlockSpec((_NB, 128), lambda i: (i, 0)),
            pl.BlockSpec((128, h), lambda i: (0, 0)),
            pl.BlockSpec((2, _NB, 16), lambda i: (0, i, 0)),
        ],
        out_specs=[
            pl.BlockSpec((2, _NB, h // 2), lambda i: (0, i, 0)),
            pl.BlockSpec((_NB, 1), lambda i: (i, 0)),
        ],
        out_shape=[
            jax.ShapeDtypeStruct((2, npad, h // 2), jnp.float32),
            jax.ShapeDtypeStruct((npad, 1), jnp.float32),
        ],
    )(xp, w1p, deg2)


def _tc_mid(agg, g, dinv, b2d, wmat, npad, split_out):
    win = agg.shape[2]
    hout = wmat.shape[1]
    grid = (npad // _NB,)

    def body(agg_ref, g_ref, dinv_ref, b_ref, w_ref, out_ref):
        dinv = dinv_ref[...]
        b = b_ref[...]
        a0 = (agg_ref[0] + g_ref[0]) * dinv + b[:, :win]
        a1 = (agg_ref[1] + g_ref[1]) * dinv + b[:, win:]
        act = jnp.maximum(jnp.concatenate([a0, a1], axis=1), 0.0)
        hh = jnp.dot(act, w_ref[...], preferred_element_type=jnp.float32)
        gg = hh * dinv
        if split_out:
            out_ref[0] = gg[:, : hout // 2]
            out_ref[1] = gg[:, hout // 2:]
        else:
            out_ref[...] = gg

    if split_out:
        out_spec = pl.BlockSpec((2, _NB, hout // 2), lambda i: (0, i, 0))
        out_shape = jax.ShapeDtypeStruct((2, npad, hout // 2), jnp.float32)
    else:
        out_spec = pl.BlockSpec((_NB, hout), lambda i: (i, 0))
        out_shape = jax.ShapeDtypeStruct((npad, hout), jnp.float32)

    return pl.pallas_call(
        body,
        grid=grid,
        in_specs=[
            pl.BlockSpec((2, _NB, win), lambda i: (0, i, 0)),
            pl.BlockSpec((2, _NB, win), lambda i: (0, i, 0)),
            pl.BlockSpec((_NB, 1), lambda i: (i, 0)),
            pl.BlockSpec((1, 2 * win), lambda i: (0, 0)),
            pl.BlockSpec((2 * win, hout), lambda i: (0, 0)),
        ],
        out_specs=out_spec,
        out_shape=out_shape,
    )(agg, g, dinv, b2d, wmat)


def _tc_final(agg2, g, dinv, b2d, npad):
    w = g.shape[1]
    grid = (npad // _NB,)

    def body(agg_ref, g_ref, dinv_ref, b_ref, out_ref):
        dinv = dinv_ref[...]
        out_ref[...] = (agg_ref[0] + agg_ref[1] + g_ref[...]) * dinv + b_ref[...]

    return pl.pallas_call(
        body,
        grid=grid,
        in_specs=[
            pl.BlockSpec((2, _NB, w), lambda i: (0, i, 0)),
            pl.BlockSpec((_NB, w), lambda i: (i, 0)),
            pl.BlockSpec((_NB, 1), lambda i: (i, 0)),
            pl.BlockSpec((1, w), lambda i: (0, 0)),
        ],
        out_specs=pl.BlockSpec((_NB, w), lambda i: (i, 0)),
        out_shape=jax.ShapeDtypeStruct((npad, w), jnp.float32),
    )(agg2, g, dinv, b2d)


def kernel(x, edge_index, W1, b1, W2, b2, W3, b3):
    n, f = x.shape
    e = edge_index.shape[1]
    h = W1.shape[1]
    c_out = W3.shape[1]
    npad = _NPAD
    nch = -(-e // _ECHUNK)
    nch = -(-nch // 32) * 32
    epad = nch * _ECHUNK

    xp = jnp.zeros((npad, 128), x.dtype).at[:n, :f].set(x)
    w1p = jnp.zeros((128, h), W1.dtype).at[:f].set(W1)
    w3p = jnp.zeros((h, 128), W3.dtype).at[:, :c_out].set(W3)
    b3p = jnp.zeros((1, 128), b3.dtype).at[0, :c_out].set(b3)
    b1r = b1.reshape(1, h)
    b2r = b2.reshape(1, h)

    fill = jnp.full((epad - e,), npad - 1, jnp.int32)
    src2d = jnp.concatenate([edge_index[0], fill]).reshape(nch, 128)
    dst2d = jnp.concatenate([edge_index[1], fill]).reshape(nch, 128)

    deg2 = _sc_degree(dst2d, npad).reshape(2, npad, 16)
    g1, dinv = _tc_layer1(xp, w1p, deg2, npad)

    agg1 = _sc_scatter(g1.reshape(2 * npad, h // 2), src2d, dst2d, npad, h // 2)
    g2 = _tc_mid(agg1.reshape(2, npad, h // 2), g1, dinv, b1r, W2, npad,
                 split_out=True)

    agg2 = _sc_scatter(g2.reshape(2 * npad, h // 2), src2d, dst2d, npad, h // 2)
    g3 = _tc_mid(agg2.reshape(2, npad, h // 2), g2, dinv, b2r, w3p, npad,
                 split_out=False)

    agg3 = _sc_scatter_esplit(g3, src2d, dst2d, npad)
    out = _tc_final(agg3.reshape(2, npad, 128), g3, dinv, b3p, npad)

    return out[:n, :c_out]
